# Initial kernel scaffold; baseline (speedup 1.0000x reference)
import jax
import jax.numpy as jnp
from jax.experimental import pallas as pl


def kernel(x, edge_index, batch, reactant_natoms, reactant_nbonds, reactant_mw, W1, b1, W2, b2, W3, b3, fc1_W, fc1_b, fc2_W, fc2_b, fc3_W, fc3_b, fc4_W, fc4_b):
    raise NotImplementedError("write your pallas kernel here")


# speedup vs baseline: 10.4985x; 10.4985x over previous
import functools

import jax
import jax.numpy as jnp
from jax import lax
from jax.experimental import pallas as pl
from jax.experimental.pallas import tpu as pltpu
from jax.experimental.pallas import tpu_sc as plsc

N = 10000
E = 320000
G = 256
D = 128

NW = 32
KC = 128
CPW = 79
EPAD = NW * CPW * KC
NPAD = N + 112
RPS = NPAD // 16

BN = 2000
GRID = N // BN

_mesh = plsc.VectorSubcoreMesh(core_axis_name="c", subcore_axis_name="s")


def _sc_agg(table, src_p, dst_p, zeros):
    @functools.partial(
        pl.kernel,
        out_type=jax.ShapeDtypeStruct((2, NPAD, D), jnp.float32),
        mesh=_mesh,
        scratch_types=[
            pltpu.VMEM((CPW, KC), jnp.int32),
            pltpu.VMEM((CPW, KC), jnp.int32),
            pltpu.VMEM((KC, D), jnp.float32),
            pltpu.VMEM_SHARED((NPAD, D), jnp.float32),
            pltpu.SemaphoreType.DMA,
        ],
    )
    def k(table_hbm, src_hbm, dst_hbm, z_hbm, out_hbm,
          src_v, dst_v, rows_v, acc, sem):
        cid = lax.axis_index("c")
        sid = lax.axis_index("s")
        wid = sid * 2 + cid
        pltpu.sync_copy(z_hbm.at[pl.ds(sid * RPS, RPS)],
                        acc.at[pl.ds(sid * RPS, RPS)])
        pltpu.sync_copy(src_hbm.at[wid], src_v)
        pltpu.sync_copy(dst_hbm.at[wid], dst_v)
        plsc.subcore_barrier()

        @pl.loop(0, CPW)
        def _(j):
            pltpu.async_copy(table_hbm.at[src_v.at[j]], rows_v, sem).wait()
            pltpu.sync_copy(rows_v, acc.at[dst_v.at[j]], add=True)

        plsc.subcore_barrier()
        pltpu.sync_copy(acc.at[pl.ds(sid * RPS, RPS)],
                        out_hbm.at[cid, pl.ds(sid * RPS, RPS)])

    return k(table, src_p, dst_p, zeros)


def _sc_deg(dst_p, zeros, ones):
    @functools.partial(
        pl.kernel,
        out_type=jax.ShapeDtypeStruct((2, NPAD, D), jnp.float32),
        mesh=_mesh,
        scratch_types=[
            pltpu.VMEM((CPW, KC), jnp.int32),
            pltpu.VMEM((KC, D), jnp.float32),
            pltpu.VMEM_SHARED((NPAD, D), jnp.float32),
            pltpu.SemaphoreType.DMA,
        ],
    )
    def k(dst_hbm, z_hbm, ones_hbm, out_hbm, dst_v, ones_v, acc, sem):
        cid = lax.axis_index("c")
        sid = lax.axis_index("s")
        wid = sid * 2 + cid
        pltpu.sync_copy(z_hbm.at[pl.ds(sid * RPS, RPS)],
                        acc.at[pl.ds(sid * RPS, RPS)])
        pltpu.sync_copy(dst_hbm.at[wid], dst_v)
        pltpu.async_copy(ones_hbm, ones_v, sem).wait()
        plsc.subcore_barrier()

        @pl.loop(0, CPW)
        def _(j):
            pltpu.sync_copy(ones_v, acc.at[dst_v.at[j]], add=True)

        plsc.subcore_barrier()
        pltpu.sync_copy(acc.at[pl.ds(sid * RPS, RPS)],
                        out_hbm.at[cid, pl.ds(sid * RPS, RPS)])

    return k(dst_p, zeros, ones)


def _mm_body(x_ref, w_ref, o_ref):
    o_ref[...] = jnp.dot(x_ref[...], w_ref[...],
                         preferred_element_type=jnp.float32)


def _mm(x, w):
    return pl.pallas_call(
        _mm_body,
        grid=(GRID,),
        in_specs=[pl.BlockSpec((BN, D), lambda i: (i, 0)),
                  pl.BlockSpec((D, D), lambda i: (0, 0))],
        out_specs=pl.BlockSpec((BN, D), lambda i: (i, 0)),
        out_shape=jax.ShapeDtypeStruct((N, D), jnp.float32),
    )(x, w)


def _dinv_of(degp_ref):
    deg = degp_ref[0, :, 0:1] + degp_ref[1, :, 0:1] + 1.0
    return lax.rsqrt(deg)


def _scale_body(degp_ref, xw_ref, o_ref):
    o_ref[...] = _dinv_of(degp_ref) * xw_ref[...]


def _scale(degp, xw):
    return pl.pallas_call(
        _scale_body,
        grid=(GRID,),
        in_specs=[pl.BlockSpec((2, BN, D), lambda i: (0, i, 0)),
                  pl.BlockSpec((BN, D), lambda i: (i, 0))],
        out_specs=pl.BlockSpec((BN, D), lambda i: (i, 0)),
        out_shape=jax.ShapeDtypeStruct((N, D), jnp.float32),
    )(degp, xw)


def _combine_body(agg_ref, degp_ref, hp_ref, b_ref, w_ref, o_ref):
    dinv = _dinv_of(degp_ref)
    p = agg_ref[0] + agg_ref[1] + hp_ref[...]
    xn = jnp.tanh(dinv * p + b_ref[...])
    o_ref[...] = dinv * jnp.dot(xn, w_ref[...],
                                preferred_element_type=jnp.float32)


def _combine(agg, degp, hp, b2d, wn):
    return pl.pallas_call(
        _combine_body,
        grid=(GRID,),
        in_specs=[pl.BlockSpec((2, BN, D), lambda i: (0, i, 0)),
                  pl.BlockSpec((2, BN, D), lambda i: (0, i, 0)),
                  pl.BlockSpec((BN, D), lambda i: (i, 0)),
                  pl.BlockSpec((1, D), lambda i: (0, 0)),
                  pl.BlockSpec((D, D), lambda i: (0, 0))],
        out_specs=pl.BlockSpec((BN, D), lambda i: (i, 0)),
        out_shape=jax.ShapeDtypeStruct((N, D), jnp.float32),
    )(agg, degp, hp, b2d, wn)


def _final_body(agg_ref, degp_ref, hp_ref, b_ref, batch_ref,
                na_ref, nb_ref, mw_ref,
                f1a_ref, f1r_ref, f1b_ref, f2w_ref, f2b_ref,
                f3w_ref, f3b_ref, f4w_ref, f4b_ref,
                o_ref, sums_ref, cnt_ref):
    i = pl.program_id(0)

    @pl.when(i == 0)
    def _():
        sums_ref[...] = jnp.zeros_like(sums_ref)
        cnt_ref[...] = jnp.zeros_like(cnt_ref)

    dinv = _dinv_of(degp_ref)
    p = agg_ref[0] + agg_ref[1] + hp_ref[...]
    h3 = dinv * p + b_ref[...]

    bb = batch_ref[0, 0, :]
    seg = lax.broadcasted_iota(jnp.int32, (G, BN), 0)
    oht = (seg == bb[None, :]).astype(jnp.float32)
    sums_ref[...] += jnp.dot(oht, h3, preferred_element_type=jnp.float32)
    cnt_ref[...] += jnp.sum(oht, axis=1, keepdims=True)

    @pl.when(i == GRID - 1)
    def _():
        cnt = cnt_ref[...]
        pooled = sums_ref[...] / jnp.maximum(cnt, 1.0)
        xg = (na_ref[...] + nb_ref[...] + mw_ref[...]) / 3.0
        xgc = jnp.where(cnt > 0, xg, 0.0)
        o = pooled @ f1a_ref[...] + xgc * f1r_ref[...] + f1b_ref[...]
        o = jnp.maximum(o, 0.0)
        o = jnp.maximum(o @ f2w_ref[...] + f2b_ref[...], 0.0)
        o = jnp.maximum(o @ f3w_ref[...] + f3b_ref[...], 0.0)
        o_ref[...] = o @ f4w_ref[...] + f4b_ref[...]


def _final(agg, degp, hp, b2d, batch3d, na, nb, mw,
           f1a, f1r, f1b, f2w, f2b, f3w, f3b, f4w, f4b):
    full = lambda shape: pl.BlockSpec(shape, lambda i: tuple(0 for _ in shape))
    return pl.pallas_call(
        _final_body,
        grid=(GRID,),
        in_specs=[pl.BlockSpec((2, BN, D), lambda i: (0, i, 0)),
                  pl.BlockSpec((2, BN, D), lambda i: (0, i, 0)),
                  pl.BlockSpec((BN, D), lambda i: (i, 0)),
                  full((1, D)),
                  pl.BlockSpec((1, 1, BN), lambda i: (i, 0, 0)),
                  full((G, 1)), full((G, 1)), full((G, 1)),
                  full((D, D)), full((1, D)), full((1, D)),
                  full((D, 64)), full((1, 64)),
                  full((64, 32)), full((1, 32)),
                  full((32, 1)), full((1, 1))],
        out_specs=full((G, 1)),
        out_shape=jax.ShapeDtypeStruct((G, 1), jnp.float32),
        scratch_shapes=[pltpu.VMEM((G, D), jnp.float32),
                        pltpu.VMEM((G, 1), jnp.float32)],
    )(agg, degp, hp, b2d, batch3d, na, nb, mw,
      f1a, f1r, f1b, f2w, f2b, f3w, f3b, f4w, f4b)


def kernel(x, edge_index, batch, reactant_natoms, reactant_nbonds,
           reactant_mw, W1, b1, W2, b2, W3, b3,
           fc1_W, fc1_b, fc2_W, fc2_b, fc3_W, fc3_b, fc4_W, fc4_b):
    src = edge_index[0]
    dst = edge_index[1]
    pad = EPAD - E
    src_p = jnp.concatenate(
        [src, jnp.zeros((pad,), jnp.int32)]).reshape(NW, CPW, KC)
    dst_p = jnp.concatenate(
        [dst, jnp.full((pad,), N, jnp.int32)]).reshape(NW, CPW, KC)
    zeros = jnp.zeros((NPAD, D), jnp.float32)
    ones = jnp.ones((KC, D), jnp.float32)

    degp = _sc_deg(dst_p, zeros, ones)
    xw1 = _mm(x, W1)
    h1p = _scale(degp, xw1)

    a1 = _sc_agg(h1p, src_p, dst_p, zeros)
    h2p = _combine(a1, degp, h1p, b1.reshape(1, D), W2)
    a2 = _sc_agg(h2p, src_p, dst_p, zeros)
    h3p = _combine(a2, degp, h2p, b2.reshape(1, D), W3)
    a3 = _sc_agg(h3p, src_p, dst_p, zeros)

    return _final(a3, degp, h3p, b3.reshape(1, D),
                  batch.reshape(GRID, 1, BN),
                  reactant_natoms.reshape(G, 1),
                  reactant_nbonds.reshape(G, 1),
                  reactant_mw.reshape(G, 1),
                  fc1_W[:D], fc1_W[D:D + 1], fc1_b.reshape(1, -1),
                  fc2_W, fc2_b.reshape(1, -1),
                  fc3_W, fc3_b.reshape(1, -1),
                  fc4_W, fc4_b.reshape(1, -1))
